# baseline (speedup 1.0000x reference)
import functools

import jax
import jax.numpy as jnp
from jax import lax
from jax.experimental import pallas as pl
from jax.experimental.pallas import tpu as pltpu
from jax.experimental.pallas import tpu_sc as plsc

_BATCH = 16384
_MF = 16
_MLP = 32
_NC = 2
_NS = 16
_NW = _NC * _NS
_BPW = _BATCH // _NW
_CH = 128
_NCH = _BPW // _CH
_L = 16


def _sc_body(uix_hbm, mix_hbm, umf_hbm, mmf_hbm, umlp_hbm, mmlp_hbm, w_hbm,
             out_hbm,
             idxu_v, idxm_v, umf_v, mmf_v, umlp_v, mmlp_v, w_v, out_v, sem):
    wid = lax.axis_index("s") * _NC + lax.axis_index("c")
    base = wid * _BPW

    pltpu.sync_copy(w_hbm, w_v)
    pltpu.sync_copy(uix_hbm.at[pl.ds(base, _BPW)], idxu_v)
    pltpu.sync_copy(mix_hbm.at[pl.ds(base, _BPW)], idxm_v)

    copies = []
    for c in range(_NCH):
        sl = pl.ds(c * _CH, _CH)
        copies.append(pltpu.async_copy(umf_hbm.at[idxu_v.at[sl]], umf_v.at[sl], sem))
        copies.append(pltpu.async_copy(mmf_hbm.at[idxm_v.at[sl]], mmf_v.at[sl], sem))
        copies.append(pltpu.async_copy(umlp_hbm.at[idxu_v.at[sl]], umlp_v.at[sl], sem))
        copies.append(pltpu.async_copy(mmlp_hbm.at[idxm_v.at[sl]], mmlp_v.at[sl], sem))
    for cp in copies:
        cp.wait()

    wvecs = [w_v[pl.ds(j * _L, _L)] for j in range(6)]
    wmf = [wvecs[0][k] for k in range(_L)]
    vu = [wvecs[1 + k // _L][k % _L] for k in range(_MLP)]
    vm = [wvecs[3 + k // _L][k % _L] for k in range(_MLP)]
    c0v = wvecs[5]
    ii = lax.iota(jnp.int32, _L)
    kvecs = [jnp.full((_L,), k, jnp.int32) for k in range(_MLP)]

    def g_body(g, carry):
        rows = g * _L + ii
        acc = c0v
        for k in range(_MF):
            u = plsc.load_gather(umf_v, [rows, kvecs[k]])
            m = plsc.load_gather(mmf_v, [rows, kvecs[k]])
            acc = acc + u * m * wmf[k]
        for k in range(_MLP):
            u = plsc.load_gather(umlp_v, [rows, kvecs[k]])
            acc = acc + u * vu[k]
        for k in range(_MLP):
            m = plsc.load_gather(mmlp_v, [rows, kvecs[k]])
            acc = acc + m * vm[k]
        out_v[pl.ds(g * _L, _L)] = 1.0 / (1.0 + jnp.exp(-acc))
        return carry

    lax.fori_loop(0, _BPW // _L, g_body, 0)
    pltpu.sync_copy(out_v, out_hbm.at[pl.ds(base, _BPW)])


_sc_call = functools.partial(
    pl.kernel,
    out_type=jax.ShapeDtypeStruct((_BATCH,), jnp.float32),
    mesh=plsc.VectorSubcoreMesh(core_axis_name="c", subcore_axis_name="s"),
    compiler_params=pltpu.CompilerParams(
        needs_layout_passes=False, use_tc_tiling_on_sc=False),
    scratch_types=[
        pltpu.VMEM((_BPW,), jnp.int32),
        pltpu.VMEM((_BPW,), jnp.int32),
        pltpu.VMEM((_BPW, _MF), jnp.float32),
        pltpu.VMEM((_BPW, _MF), jnp.float32),
        pltpu.VMEM((_BPW, _MLP), jnp.float32),
        pltpu.VMEM((_BPW, _MLP), jnp.float32),
        pltpu.VMEM((3 * _MLP,), jnp.float32),
        pltpu.VMEM((_BPW,), jnp.float32),
        pltpu.SemaphoreType.DMA,
    ],
)(_sc_body)


def kernel(X, user_mf, movie_mf, user_mlp, movie_mlp, W1, b1, W2, b2, Wf, bf):
    Xi = X.astype(jnp.int32)
    uix = Xi[:, 0]
    mix = Xi[:, 1]
    wf = Wf[0]
    wf_out = wf[_MF:]
    t = wf_out @ W2
    v = t @ W1
    c0 = jnp.dot(t, b1) + jnp.dot(wf_out, b2) + bf[0]
    wpack = jnp.concatenate(
        [wf[:_MF], v, jnp.full((_L,), c0, jnp.float32)])
    out = _sc_call(uix, mix, user_mf, movie_mf, user_mlp, movie_mlp, wpack)
    return out.reshape(_BATCH, 1)
